# Initial kernel scaffold; baseline (speedup 1.0000x reference)
import jax
import jax.numpy as jnp
from jax.experimental import pallas as pl


def kernel(logits, scores, step):
    raise NotImplementedError("write your pallas kernel here")


# speedup vs baseline: 1.9675x; 1.9675x over previous
import functools

import jax
import jax.numpy as jnp
from jax.experimental import pallas as pl
import jax.experimental.pallas.tpu as pltpu

BSZ = 32
BEAM = 5
VOCAB = 100000
PAD = 1
EOS = 2
MIN_LEN = 1
K = 10
ROWS = BSZ * BEAM
GROUP = 8
NGROUP = ROWS // GROUP
LANES = 128
VP = ((VOCAB + LANES - 1) // LANES) * LANES
NCHUNK = VP // LANES
NEG = float("-inf")
IMAX = 2**31 - 1


def _scan_kernel(x_ref, adj_ref, eos_ref, val_ref, idx_ref, scratch_ref):
    x = x_ref[...]
    m = jnp.max(x, axis=1, keepdims=True)
    s = jnp.sum(jnp.exp(x - m), axis=1, keepdims=True)
    c = adj_ref[...] - m - jnp.log(s)

    scratch_ref[...] = jnp.full((GROUP, VP), NEG, jnp.float32)
    scratch_ref[:, :VOCAB] = x + c
    scratch_ref[:, PAD:PAD + 1] = jnp.full((GROUP, 1), NEG, jnp.float32)
    scratch_ref[:, EOS:EOS + 1] = x[:, EOS:EOS + 1] + c + eos_ref[...]

    g = pl.program_id(0)
    row = g * GROUP + jax.lax.broadcasted_iota(jnp.int32, (GROUP, LANES), 0)
    beam = row % BEAM
    base = beam * VOCAB + jax.lax.broadcasted_iota(
        jnp.int32, (GROUP, LANES), 1)

    t0 = jnp.full((GROUP, LANES), NEG, jnp.float32)
    i0 = jnp.full((GROUP, LANES), IMAX, jnp.int32)
    init = (tuple(t0 for _ in range(K)), tuple(i0 for _ in range(K)))

    def body(j, carry):
        ts, tis = carry
        off = pl.multiple_of(j * LANES, LANES)
        v = scratch_ref[:, pl.ds(off, LANES)]
        vi = base + j * LANES
        nts, ntis = [], []
        for k in range(K):
            t, ti = ts[k], tis[k]
            ge = v > t
            nts.append(jnp.where(ge, v, t))
            ntis.append(jnp.where(ge, vi, ti))
            v, vi = jnp.where(ge, t, v), jnp.where(ge, ti, vi)
        return (tuple(nts), tuple(ntis))

    ts, tis = jax.lax.fori_loop(0, NCHUNK, body, init)
    for k in range(K):
        val_ref[:, k * LANES:(k + 1) * LANES] = ts[k]
        idx_ref[:, k * LANES:(k + 1) * LANES] = tis[k]


def _merge_kernel(val_ref, idx_ref, sc_ref, tok_ref, beam_ref):
    ts = [val_ref[0, :, k * LANES:(k + 1) * LANES] for k in range(K)]
    tis = [idx_ref[0, :, k * LANES:(k + 1) * LANES] for k in range(K)]
    for ko in range(K):
        t0, i0 = ts[0], tis[0]
        gm = jnp.max(t0)
        eqm = t0 == gm
        im = jnp.min(jnp.where(eqm, i0, IMAX))
        sel = eqm & (i0 == im)
        sc_ref[0:1, 0:1, ko:ko + 1] = gm.reshape(1, 1, 1)
        tok_ref[0:1, 0:1, ko:ko + 1] = (im % VOCAB).reshape(1, 1, 1)
        beam_ref[0:1, 0:1, ko:ko + 1] = (im // VOCAB).reshape(1, 1, 1)
        nts = [jnp.where(sel, ts[k + 1], ts[k]) for k in range(K - 1)]
        ntis = [jnp.where(sel, tis[k + 1], tis[k]) for k in range(K - 1)]
        nts.append(jnp.where(sel, NEG, ts[K - 1]))
        ntis.append(jnp.where(sel, IMAX, tis[K - 1]))
        ts, tis = nts, ntis


@functools.partial(jax.jit, static_argnames=())
def kernel(logits, scores, step):
    step = jnp.asarray(step)
    beam = jnp.arange(ROWS, dtype=jnp.int32) % BEAM
    adj = jnp.where(step == 0,
                    jnp.where(beam == 0, 0.0, -jnp.inf),
                    scores).astype(jnp.float32).reshape(ROWS, 1)
    eos_add = jnp.where(step < MIN_LEN, -jnp.inf, 0.0).astype(
        jnp.float32).reshape(1, 1)

    vals, idxs = pl.pallas_call(
        _scan_kernel,
        grid=(NGROUP,),
        in_specs=[
            pl.BlockSpec((GROUP, VOCAB), lambda g: (g, 0)),
            pl.BlockSpec((GROUP, 1), lambda g: (g, 0)),
            pl.BlockSpec((1, 1), lambda g: (0, 0)),
        ],
        out_specs=[
            pl.BlockSpec((GROUP, K * LANES), lambda g: (g, 0)),
            pl.BlockSpec((GROUP, K * LANES), lambda g: (g, 0)),
        ],
        out_shape=[
            jax.ShapeDtypeStruct((ROWS, K * LANES), jnp.float32),
            jax.ShapeDtypeStruct((ROWS, K * LANES), jnp.int32),
        ],
        scratch_shapes=[pltpu.VMEM((GROUP, VP), jnp.float32)],
    )(logits, adj, eos_add)

    vals3 = vals.reshape(BSZ, BEAM, K * LANES)
    idxs3 = idxs.reshape(BSZ, BEAM, K * LANES)

    sc, tok, bm = pl.pallas_call(
        _merge_kernel,
        grid=(BSZ,),
        in_specs=[
            pl.BlockSpec((1, BEAM, K * LANES), lambda b: (b, 0, 0)),
            pl.BlockSpec((1, BEAM, K * LANES), lambda b: (b, 0, 0)),
        ],
        out_specs=[
            pl.BlockSpec((1, 1, K), lambda b: (b, 0, 0)),
            pl.BlockSpec((1, 1, K), lambda b: (b, 0, 0)),
            pl.BlockSpec((1, 1, K), lambda b: (b, 0, 0)),
        ],
        out_shape=[
            jax.ShapeDtypeStruct((BSZ, 1, K), jnp.float32),
            jax.ShapeDtypeStruct((BSZ, 1, K), jnp.int32),
            jax.ShapeDtypeStruct((BSZ, 1, K), jnp.int32),
        ],
    )(vals3, idxs3)

    return sc.reshape(BSZ, K), tok.reshape(BSZ, K), bm.reshape(BSZ, K)


# speedup vs baseline: 3.0384x; 1.5443x over previous
import functools

import jax
import jax.numpy as jnp
from jax.experimental import pallas as pl
import jax.experimental.pallas.tpu as pltpu

BSZ = 32
BEAM = 5
VOCAB = 100000
PAD = 1
EOS = 2
MIN_LEN = 1
K = 10
ROWS = BSZ * BEAM
GROUP = 8
NGROUP = ROWS // GROUP
W = 256
NFULL = VOCAB // W
TAIL = VOCAB - NFULL * W
MB = 8
NMERGE = BSZ // MB
LW = BEAM * W
NEG = float("-inf")
IMAX = 2**31 - 1
SLICES = (0, 25088, 50176, 75264, VOCAB)


def _insert(v, vi, ts, tis):
    nts, ntis = [], []
    for k in range(K):
        t, ti = ts[k], tis[k]
        ge = v > t
        nts.append(jnp.maximum(t, v))
        ntis.append(jnp.where(ge, vi, ti))
        v = jnp.minimum(t, v)
        vi = jnp.where(ge, ti, vi)
    return nts, ntis


def _scan_kernel(x_ref, padeos_ref, adj_ref, val_ref, idx_ref, c_ref):
    x = x_ref[...]
    ms = [jnp.max(x[:, SLICES[i]:SLICES[i + 1]], axis=1, keepdims=True)
          for i in range(4)]
    m = jnp.maximum(jnp.maximum(ms[0], ms[1]), jnp.maximum(ms[2], ms[3]))
    ss = [jnp.sum(jnp.exp(x[:, SLICES[i]:SLICES[i + 1]] - m), axis=1,
                  keepdims=True) for i in range(4)]
    s = (ss[0] + ss[1]) + (ss[2] + ss[3])
    c_ref[...] = adj_ref[...] - m - jnp.log(s)

    lane = jax.lax.broadcasted_iota(jnp.int32, (GROUP, W), 1)

    ts = [jnp.full((GROUP, W), NEG, jnp.float32) for _ in range(K)]
    tis = [jnp.full((GROUP, W), IMAX, jnp.int32) for _ in range(K)]

    v0 = x_ref[:, :W] + padeos_ref[...]
    ts, tis = _insert(v0, lane, ts, tis)

    def body(j, carry):
        ts, tis = carry
        off = pl.multiple_of(j * W, W)
        v = x_ref[:, pl.ds(off, W)]
        nts, ntis = _insert(v, lane + j * W, tuple(ts), tuple(tis))
        return tuple(nts), tuple(ntis)

    ts, tis = jax.lax.fori_loop(1, NFULL, body, (tuple(ts), tuple(tis)))
    ts, tis = list(ts), list(tis)

    vt = jnp.concatenate(
        [x_ref[:, NFULL * W:VOCAB],
         jnp.full((GROUP, W - TAIL), NEG, jnp.float32)], axis=1)
    ts, tis = _insert(vt, lane + NFULL * W, ts, tis)

    for k in range(K):
        val_ref[:, k * W:(k + 1) * W] = ts[k]
        idx_ref[:, k * W:(k + 1) * W] = tis[k]


def _merge_kernel(val_ref, idx_ref, c_ref, cb_ref, sc_ref, tok_ref,
                  beam_ref):
    c = c_ref[...]
    cb = cb_ref[...]
    ts = [val_ref[:, k * LW:(k + 1) * LW] + c for k in range(K)]
    tis = [idx_ref[:, k * LW:(k + 1) * LW] + cb for k in range(K)]
    for ko in range(K):
        t0, i0 = ts[0], tis[0]
        gm = jnp.max(t0, axis=1, keepdims=True)
        eqm = t0 == gm
        im = jnp.min(jnp.where(eqm, i0, IMAX), axis=1, keepdims=True)
        sel = eqm & (i0 == im)
        sc_ref[:, ko:ko + 1] = gm
        tok_ref[:, ko:ko + 1] = im % VOCAB
        beam_ref[:, ko:ko + 1] = im // VOCAB
        nts = [jnp.where(sel, ts[k + 1], ts[k]) for k in range(K - 1)]
        ntis = [jnp.where(sel, tis[k + 1], tis[k]) for k in range(K - 1)]
        nts.append(jnp.where(sel, NEG, ts[K - 1]))
        ntis.append(jnp.where(sel, IMAX, tis[K - 1]))
        ts, tis = nts, ntis


@functools.partial(jax.jit, static_argnames=())
def kernel(logits, scores, step):
    step = jnp.asarray(step)
    beam = jnp.arange(ROWS, dtype=jnp.int32) % BEAM
    adj = jnp.where(step == 0,
                    jnp.where(beam == 0, 0.0, -jnp.inf),
                    scores).astype(jnp.float32).reshape(ROWS, 1)
    eos_add = jnp.where(step < MIN_LEN, -jnp.inf, 0.0).astype(jnp.float32)
    lane0 = jnp.arange(W)
    padeos = (jnp.where(lane0 == PAD, -jnp.inf, 0.0).astype(jnp.float32)
              + jnp.where(lane0 == EOS, 1.0, 0.0) * eos_add).reshape(1, W)

    vals, idxs, c = pl.pallas_call(
        _scan_kernel,
        grid=(NGROUP,),
        in_specs=[
            pl.BlockSpec((GROUP, VOCAB), lambda g: (g, 0)),
            pl.BlockSpec((1, W), lambda g: (0, 0)),
            pl.BlockSpec((GROUP, 1), lambda g: (g, 0)),
        ],
        out_specs=[
            pl.BlockSpec((GROUP, K * W), lambda g: (g, 0)),
            pl.BlockSpec((GROUP, K * W), lambda g: (g, 0)),
            pl.BlockSpec((GROUP, 1), lambda g: (g, 0)),
        ],
        out_shape=[
            jax.ShapeDtypeStruct((ROWS, K * W), jnp.float32),
            jax.ShapeDtypeStruct((ROWS, K * W), jnp.int32),
            jax.ShapeDtypeStruct((ROWS, 1), jnp.float32),
        ],
    )(logits, padeos, adj)

    vals_m = vals.reshape(BSZ, BEAM, K, W).transpose(0, 2, 1, 3).reshape(
        BSZ, K * LW)
    idxs_m = idxs.reshape(BSZ, BEAM, K, W).transpose(0, 2, 1, 3).reshape(
        BSZ, K * LW)
    c_m = jnp.broadcast_to(c.reshape(BSZ, BEAM, 1),
                           (BSZ, BEAM, W)).reshape(BSZ, LW)
    cb = (jnp.arange(LW, dtype=jnp.int32) // W * VOCAB).reshape(1, LW)

    sc, tok, bm = pl.pallas_call(
        _merge_kernel,
        grid=(NMERGE,),
        in_specs=[
            pl.BlockSpec((MB, K * LW), lambda b: (b, 0)),
            pl.BlockSpec((MB, K * LW), lambda b: (b, 0)),
            pl.BlockSpec((MB, LW), lambda b: (b, 0)),
            pl.BlockSpec((1, LW), lambda b: (0, 0)),
        ],
        out_specs=[
            pl.BlockSpec((MB, K), lambda b: (b, 0)),
            pl.BlockSpec((MB, K), lambda b: (b, 0)),
            pl.BlockSpec((MB, K), lambda b: (b, 0)),
        ],
        out_shape=[
            jax.ShapeDtypeStruct((BSZ, K), jnp.float32),
            jax.ShapeDtypeStruct((BSZ, K), jnp.int32),
            jax.ShapeDtypeStruct((BSZ, K), jnp.int32),
        ],
    )(vals_m, idxs_m, c_m, cb)

    return sc, tok, bm


# speedup vs baseline: 4.7445x; 1.5615x over previous
import functools

import jax
import jax.numpy as jnp
from jax.experimental import pallas as pl

BSZ = 32
BEAM = 5
VOCAB = 100000
PAD = 1
EOS = 2
MIN_LEN = 1
K = 10
ROWS = BSZ * BEAM
GROUP = 8
NGROUP = ROWS // GROUP
W = 256
NFULL = VOCAB // W
TAIL = VOCAB - NFULL * W
MB = 8
NMERGE = BSZ // MB
LW = BEAM * W
NEG = float("-inf")
IMAX = 2**31 - 1
SLICES = (0, 25088, 50176, 75264, VOCAB)


def _insert(v, vi, ts, tis, depth):
    nts, ntis = [], []
    for k in range(depth):
        t, ti = ts[k], tis[k]
        ge = v > t
        nts.append(jnp.maximum(t, v))
        ntis.append(jnp.where(ge, vi, ti))
        v = jnp.minimum(t, v)
        vi = jnp.where(ge, ti, vi)
    return nts, ntis


def _scan_kernel(x_ref, padeos_ref, adj_ref, val_ref, idx_ref, c_ref, *,
                 depth):
    x = x_ref[...]
    ms = [jnp.max(x[:, SLICES[i]:SLICES[i + 1]], axis=1, keepdims=True)
          for i in range(4)]
    m = jnp.maximum(jnp.maximum(ms[0], ms[1]), jnp.maximum(ms[2], ms[3]))
    ss = [jnp.sum(jnp.exp(x[:, SLICES[i]:SLICES[i + 1]] - m), axis=1,
                  keepdims=True) for i in range(4)]
    s = (ss[0] + ss[1]) + (ss[2] + ss[3])
    c_ref[...] = adj_ref[...] - m - jnp.log(s)

    lane = jax.lax.broadcasted_iota(jnp.int32, (GROUP, W), 1)

    ts = [jnp.full((GROUP, W), NEG, jnp.float32) for _ in range(depth)]
    tis = [jnp.full((GROUP, W), IMAX, jnp.int32) for _ in range(depth)]

    v0 = x_ref[:, :W] + padeos_ref[...]
    ts, tis = _insert(v0, lane, ts, tis, depth)

    def body(j, carry):
        ts, tis = carry
        off = pl.multiple_of(j * W, W)
        v = x_ref[:, pl.ds(off, W)]
        nts, ntis = _insert(v, lane + j * W, ts, tis, depth)
        return tuple(nts), tuple(ntis)

    ts, tis = jax.lax.fori_loop(1, NFULL, body, (tuple(ts), tuple(tis)))
    ts, tis = list(ts), list(tis)

    vt = jnp.concatenate(
        [x_ref[:, NFULL * W:VOCAB],
         jnp.full((GROUP, W - TAIL), NEG, jnp.float32)], axis=1)
    ts, tis = _insert(vt, lane + NFULL * W, ts, tis, depth)

    for k in range(depth):
        val_ref[:, k * W:(k + 1) * W] = ts[k]
        idx_ref[:, k * W:(k + 1) * W] = tis[k]


def _merge_kernel(val_ref, idx_ref, c_ref, cb_ref, sc_ref, tok_ref,
                  beam_ref, flag_ref, *, depth):
    c = c_ref[...]
    cb = cb_ref[...]
    ts = [val_ref[:, k * LW:(k + 1) * LW] + c for k in range(depth)]
    tis = [idx_ref[:, k * LW:(k + 1) * LW] + cb for k in range(depth)]
    deepest = ts[depth - 1]
    gm = None
    for ko in range(K):
        t0, i0 = ts[0], tis[0]
        gm = jnp.max(t0, axis=1, keepdims=True)
        eqm = t0 == gm
        im = jnp.min(jnp.where(eqm, i0, IMAX), axis=1, keepdims=True)
        sel = eqm & (i0 == im)
        sc_ref[:, ko:ko + 1] = gm
        tok_ref[:, ko:ko + 1] = im % VOCAB
        beam_ref[:, ko:ko + 1] = im // VOCAB
        nts = [jnp.where(sel, ts[k + 1], ts[k]) for k in range(depth - 1)]
        ntis = [jnp.where(sel, tis[k + 1], tis[k])
                for k in range(depth - 1)]
        nts.append(jnp.where(sel, NEG, ts[depth - 1]))
        ntis.append(jnp.where(sel, IMAX, tis[depth - 1]))
        ts, tis = nts, ntis
    flag_ref[...] = jnp.max(
        jnp.where(deepest >= gm, 1, 0).astype(jnp.int32),
        axis=1, keepdims=True)


def _run(logits, padeos, adj, depth):
    vals, idxs, c = pl.pallas_call(
        functools.partial(_scan_kernel, depth=depth),
        grid=(NGROUP,),
        in_specs=[
            pl.BlockSpec((GROUP, VOCAB), lambda g: (g, 0)),
            pl.BlockSpec((1, W), lambda g: (0, 0)),
            pl.BlockSpec((GROUP, 1), lambda g: (g, 0)),
        ],
        out_specs=[
            pl.BlockSpec((GROUP, depth * W), lambda g: (g, 0)),
            pl.BlockSpec((GROUP, depth * W), lambda g: (g, 0)),
            pl.BlockSpec((GROUP, 1), lambda g: (g, 0)),
        ],
        out_shape=[
            jax.ShapeDtypeStruct((ROWS, depth * W), jnp.float32),
            jax.ShapeDtypeStruct((ROWS, depth * W), jnp.int32),
            jax.ShapeDtypeStruct((ROWS, 1), jnp.float32),
        ],
    )(logits, padeos, adj)

    vals_m = vals.reshape(BSZ, BEAM, depth, W).transpose(
        0, 2, 1, 3).reshape(BSZ, depth * LW)
    idxs_m = idxs.reshape(BSZ, BEAM, depth, W).transpose(
        0, 2, 1, 3).reshape(BSZ, depth * LW)
    c_m = jnp.broadcast_to(c.reshape(BSZ, BEAM, 1),
                           (BSZ, BEAM, W)).reshape(BSZ, LW)
    cb = (jnp.arange(LW, dtype=jnp.int32) // W * VOCAB).reshape(1, LW)

    sc, tok, bm, flag = pl.pallas_call(
        functools.partial(_merge_kernel, depth=depth),
        grid=(NMERGE,),
        in_specs=[
            pl.BlockSpec((MB, depth * LW), lambda b: (b, 0)),
            pl.BlockSpec((MB, depth * LW), lambda b: (b, 0)),
            pl.BlockSpec((MB, LW), lambda b: (b, 0)),
            pl.BlockSpec((1, LW), lambda b: (0, 0)),
        ],
        out_specs=[
            pl.BlockSpec((MB, K), lambda b: (b, 0)),
            pl.BlockSpec((MB, K), lambda b: (b, 0)),
            pl.BlockSpec((MB, K), lambda b: (b, 0)),
            pl.BlockSpec((MB, 1), lambda b: (b, 0)),
        ],
        out_shape=[
            jax.ShapeDtypeStruct((BSZ, K), jnp.float32),
            jax.ShapeDtypeStruct((BSZ, K), jnp.int32),
            jax.ShapeDtypeStruct((BSZ, K), jnp.int32),
            jax.ShapeDtypeStruct((BSZ, 1), jnp.int32),
        ],
    )(vals_m, idxs_m, c_m, cb)
    return sc, tok, bm, flag


@functools.partial(jax.jit, static_argnames=())
def kernel(logits, scores, step):
    step = jnp.asarray(step)
    beam = jnp.arange(ROWS, dtype=jnp.int32) % BEAM
    adj = jnp.where(step == 0,
                    jnp.where(beam == 0, 0.0, -jnp.inf),
                    scores).astype(jnp.float32).reshape(ROWS, 1)
    eos_add = jnp.where(step < MIN_LEN, -jnp.inf, 0.0).astype(jnp.float32)
    lane0 = jnp.arange(W)
    padeos = (jnp.where(lane0 == PAD, -jnp.inf, 0.0)
              + jnp.where(lane0 == EOS, eos_add, 0.0)).astype(
                  jnp.float32).reshape(1, W)

    sc, tok, bm, flag = _run(logits, padeos, adj, 5)
    return jax.lax.cond(
        jnp.any(flag > 0),
        lambda: _run(logits, padeos, adj, K)[:3],
        lambda: (sc, tok, bm),
    )


# speedup vs baseline: 5.1417x; 1.0837x over previous
import functools

import jax
import jax.numpy as jnp
from jax.experimental import pallas as pl

BSZ = 32
BEAM = 5
VOCAB = 100000
PAD = 1
EOS = 2
MIN_LEN = 1
K = 10
ROWS = BSZ * BEAM
GROUP = 8
NGROUP = ROWS // GROUP
W = 256
NFULL = VOCAB // W
TAIL = VOCAB - NFULL * W
MB = 8
NMERGE = BSZ // MB
LW = BEAM * W
NEG = float("-inf")
IMAX = 2**31 - 1
SLICES = (0, 25088, 50176, 75264, VOCAB)


def _insert(v, vi, ts, tis, depth):
    ge = [v > ts[k] for k in range(depth)]
    nts = [jnp.where(ge[0], v, ts[0])]
    ntis = [jnp.where(ge[0], vi, tis[0])]
    for k in range(1, depth):
        nts.append(jnp.where(ge[k], jnp.where(ge[k - 1], ts[k - 1], v),
                             ts[k]))
        ntis.append(jnp.where(ge[k], jnp.where(ge[k - 1], tis[k - 1], vi),
                              tis[k]))
    return nts, ntis


def _scan_kernel(x_ref, padeos_ref, adj_ref, val_ref, idx_ref, c_ref, *,
                 depth):
    x = x_ref[...]
    ms = [jnp.max(x[:, SLICES[i]:SLICES[i + 1]], axis=1, keepdims=True)
          for i in range(4)]
    m = jnp.maximum(jnp.maximum(ms[0], ms[1]), jnp.maximum(ms[2], ms[3]))
    ss = [jnp.sum(jnp.exp(x[:, SLICES[i]:SLICES[i + 1]] - m), axis=1,
                  keepdims=True) for i in range(4)]
    s = (ss[0] + ss[1]) + (ss[2] + ss[3])
    c_ref[...] = adj_ref[...] - m - jnp.log(s)

    lane = jax.lax.broadcasted_iota(jnp.int32, (GROUP, W), 1)

    ts = [jnp.full((GROUP, W), NEG, jnp.float32) for _ in range(depth)]
    tis = [jnp.full((GROUP, W), IMAX, jnp.int32) for _ in range(depth)]

    v0 = x_ref[:, :W] + padeos_ref[...]
    ts, tis = _insert(v0, lane, ts, tis, depth)

    def body(j, carry):
        ts, tis = carry
        off = pl.multiple_of(j * W, W)
        v = x_ref[:, pl.ds(off, W)]
        nts, ntis = _insert(v, lane + j * W, ts, tis, depth)
        return tuple(nts), tuple(ntis)

    ts, tis = jax.lax.fori_loop(1, NFULL, body, (tuple(ts), tuple(tis)))
    ts, tis = list(ts), list(tis)

    vt = jnp.concatenate(
        [x_ref[:, NFULL * W:VOCAB],
         jnp.full((GROUP, W - TAIL), NEG, jnp.float32)], axis=1)
    ts, tis = _insert(vt, lane + NFULL * W, ts, tis, depth)

    for k in range(depth):
        val_ref[:, k * W:(k + 1) * W] = ts[k]
        idx_ref[:, k * W:(k + 1) * W] = tis[k]


def _merge_kernel(val_ref, idx_ref, c_ref, cb_ref, sc_ref, tok_ref,
                  beam_ref, flag_ref, *, depth):
    c = c_ref[...]
    cb = cb_ref[...]
    ts = [val_ref[:, k * LW:(k + 1) * LW] + c for k in range(depth)]
    tis = [idx_ref[:, k * LW:(k + 1) * LW] + cb for k in range(depth)]
    deepest = ts[depth - 1]
    gm = None
    for ko in range(K):
        t0, i0 = ts[0], tis[0]
        gm = jnp.max(t0, axis=1, keepdims=True)
        eqm = t0 == gm
        im = jnp.min(jnp.where(eqm, i0, IMAX), axis=1, keepdims=True)
        sel = eqm & (i0 == im)
        sc_ref[:, ko:ko + 1] = gm
        tok_ref[:, ko:ko + 1] = im % VOCAB
        beam_ref[:, ko:ko + 1] = im // VOCAB
        nts = [jnp.where(sel, ts[k + 1], ts[k]) for k in range(depth - 1)]
        ntis = [jnp.where(sel, tis[k + 1], tis[k])
                for k in range(depth - 1)]
        nts.append(jnp.where(sel, NEG, ts[depth - 1]))
        ntis.append(jnp.where(sel, IMAX, tis[depth - 1]))
        ts, tis = nts, ntis
    flag_ref[...] = jnp.max(
        jnp.where(deepest >= gm, 1, 0).astype(jnp.int32),
        axis=1, keepdims=True)


def _run(logits, padeos, adj, depth):
    vals, idxs, c = pl.pallas_call(
        functools.partial(_scan_kernel, depth=depth),
        grid=(NGROUP,),
        in_specs=[
            pl.BlockSpec((GROUP, VOCAB), lambda g: (g, 0)),
            pl.BlockSpec((1, W), lambda g: (0, 0)),
            pl.BlockSpec((GROUP, 1), lambda g: (g, 0)),
        ],
        out_specs=[
            pl.BlockSpec((GROUP, depth * W), lambda g: (g, 0)),
            pl.BlockSpec((GROUP, depth * W), lambda g: (g, 0)),
            pl.BlockSpec((GROUP, 1), lambda g: (g, 0)),
        ],
        out_shape=[
            jax.ShapeDtypeStruct((ROWS, depth * W), jnp.float32),
            jax.ShapeDtypeStruct((ROWS, depth * W), jnp.int32),
            jax.ShapeDtypeStruct((ROWS, 1), jnp.float32),
        ],
    )(logits, padeos, adj)

    vals_m = vals.reshape(BSZ, BEAM, depth, W).transpose(
        0, 2, 1, 3).reshape(BSZ, depth * LW)
    idxs_m = idxs.reshape(BSZ, BEAM, depth, W).transpose(
        0, 2, 1, 3).reshape(BSZ, depth * LW)
    c_m = jnp.broadcast_to(c.reshape(BSZ, BEAM, 1),
                           (BSZ, BEAM, W)).reshape(BSZ, LW)
    cb = (jnp.arange(LW, dtype=jnp.int32) // W * VOCAB).reshape(1, LW)

    sc, tok, bm, flag = pl.pallas_call(
        functools.partial(_merge_kernel, depth=depth),
        grid=(NMERGE,),
        in_specs=[
            pl.BlockSpec((MB, depth * LW), lambda b: (b, 0)),
            pl.BlockSpec((MB, depth * LW), lambda b: (b, 0)),
            pl.BlockSpec((MB, LW), lambda b: (b, 0)),
            pl.BlockSpec((1, LW), lambda b: (0, 0)),
        ],
        out_specs=[
            pl.BlockSpec((MB, K), lambda b: (b, 0)),
            pl.BlockSpec((MB, K), lambda b: (b, 0)),
            pl.BlockSpec((MB, K), lambda b: (b, 0)),
            pl.BlockSpec((MB, 1), lambda b: (b, 0)),
        ],
        out_shape=[
            jax.ShapeDtypeStruct((BSZ, K), jnp.float32),
            jax.ShapeDtypeStruct((BSZ, K), jnp.int32),
            jax.ShapeDtypeStruct((BSZ, K), jnp.int32),
            jax.ShapeDtypeStruct((BSZ, 1), jnp.int32),
        ],
    )(vals_m, idxs_m, c_m, cb)
    return sc, tok, bm, flag


@functools.partial(jax.jit, static_argnames=())
def kernel(logits, scores, step):
    step = jnp.asarray(step)
    beam = jnp.arange(ROWS, dtype=jnp.int32) % BEAM
    adj = jnp.where(step == 0,
                    jnp.where(beam == 0, 0.0, -jnp.inf),
                    scores).astype(jnp.float32).reshape(ROWS, 1)
    eos_add = jnp.where(step < MIN_LEN, -jnp.inf, 0.0).astype(jnp.float32)
    lane0 = jnp.arange(W)
    padeos = (jnp.where(lane0 == PAD, -jnp.inf, 0.0)
              + jnp.where(lane0 == EOS, eos_add, 0.0)).astype(
                  jnp.float32).reshape(1, W)

    sc, tok, bm, flag = _run(logits, padeos, adj, 5)
    return jax.lax.cond(
        jnp.any(flag > 0),
        lambda: _run(logits, padeos, adj, K)[:3],
        lambda: (sc, tok, bm),
    )
